# baseline (speedup 1.0000x reference)
import functools

import jax
import jax.numpy as jnp
from jax import lax
from jax.experimental import pallas as pl
from jax.experimental.pallas import tpu as pltpu
from jax.experimental.pallas import tpu_sc as plsc


def _gather_kernel(B, F, V, D):
    info = plsc.get_sparse_core_info()
    NC, NS, L = info.num_cores, info.num_subcores, info.num_lanes
    NW = NC * NS
    R = B * F
    assert R % NW == 0
    rpw = R // NW
    assert rpw % L == 0 and rpw % 8 == 0

    mesh = plsc.VectorSubcoreMesh(core_axis_name="c", subcore_axis_name="s")

    @functools.partial(
        pl.kernel,
        mesh=mesh,
        out_type=jax.ShapeDtypeStruct((R, D), jnp.float32),
        compiler_params=pltpu.CompilerParams(use_tc_tiling_on_sc=False),
        scratch_types=[
            pltpu.VMEM((rpw,), jnp.int32),
            pltpu.VMEM((rpw,), jnp.int32),
            pltpu.VMEM((rpw, D), jnp.float32),
            pltpu.SemaphoreType.DMA,
        ],
    )
    def k(idx_hbm, off_hbm, tab_hbm, out_hbm, idx_v, off_v, rows_v, sem):
        wid = lax.axis_index("s") * NC + lax.axis_index("c")
        base = wid * rpw
        pltpu.sync_copy(idx_hbm.at[pl.ds(base, rpw)], idx_v)
        pltpu.sync_copy(off_hbm, off_v)

        def add_off(i, carry):
            sl = pl.ds(i * L, L)
            idx_v[sl] = idx_v[sl] + off_v[sl]
            return carry

        lax.fori_loop(0, rpw // L, add_off, 0)
        pltpu.async_copy(tab_hbm.at[idx_v], rows_v, sem).wait()
        pltpu.sync_copy(rows_v, out_hbm.at[pl.ds(base, rpw)])

    return k


def kernel(inputs, tables):
    B, F = inputs.shape
    _, V, D = tables.shape
    R = B * F
    flat_idx = inputs.reshape(R)
    flat_tab = tables.reshape(F * V, D)
    rpw = R // 32
    offsets = jnp.tile(jnp.arange(F, dtype=jnp.int32) * V, rpw // F)
    out_flat = _gather_kernel(B, F, V, D)(flat_idx, offsets, flat_tab)
    return out_flat.reshape(B, F, D)
